# baseline (speedup 1.0000x reference)
import functools

import jax
import jax.numpy as jnp
from jax import lax
from jax.experimental import pallas as pl
from jax.experimental.pallas import tpu as pltpu
from jax.experimental.pallas import tpu_sc as plsc

BLK = 2048

_SC_WORKERS = 32


def _sc_override(cu_ref, out_ref, o_v, idx_v):
    slc = o_v.shape[0]
    wid = lax.axis_index("s") * 2 + lax.axis_index("c")
    base = wid * slc
    pltpu.sync_copy(cu_ref.at[pl.ds(0, 16)], idx_v)
    zeros16 = jnp.zeros((16,), jnp.float32)
    for i in range(slc // 16):
        o_v[pl.ds(i * 16, 16)] = zeros16
    idx = idx_v[...]
    in_range = jnp.logical_and(idx >= base, idx < base + slc)
    loc = jnp.where(in_range, idx - base, 0)
    plsc.store_scatter(o_v, [loc], jnp.full((16,), 1.0, jnp.float32),
                       mask=in_range)
    pltpu.sync_copy(o_v, out_ref.at[pl.ds(base, slc)])


def _build_override(cu_seqlens, T):
    mesh = plsc.VectorSubcoreMesh(core_axis_name="c", subcore_axis_name="s")
    fn = pl.kernel(
        _sc_override,
        out_type=jax.ShapeDtypeStruct((T,), jnp.float32),
        mesh=mesh,
        scratch_types=[
            pltpu.VMEM((T // _SC_WORKERS,), jnp.float32),
            pltpu.VMEM((16,), jnp.int32),
        ],
        compiler_params=pltpu.CompilerParams(needs_layout_passes=False),
    )
    return fn(cu_seqlens)


def _routing_block(scal_ref, hs_ref, wq_ref, wk_ref, ov_ref,
                   p_ref, carry_ref):
    b = pl.program_id(0)
    cur = hs_ref[...]

    dims = (((1,), (1,)), ((), ()))
    a = lax.dot_general(cur, wq_ref[...], dims,
                        preferred_element_type=jnp.float32)
    k = lax.dot_general(cur, wk_ref[...], dims,
                        preferred_element_type=jnp.float32)

    prev_last = carry_ref[7:8, :]
    carry_ref[...] = a[BLK - 8:, :]
    row_iota = lax.broadcasted_iota(jnp.int32, (BLK, 1), 0)
    q = jnp.where(row_iota == 0, prev_last, pltpu.roll(a, shift=1, axis=0))

    dot = jnp.sum(q * k, axis=1, keepdims=True)
    qn = jnp.maximum(jnp.sqrt(jnp.sum(q * q, axis=1, keepdims=True)), 1e-12)
    kn = jnp.maximum(jnp.sqrt(jnp.sum(k * k, axis=1, keepdims=True)), 1e-12)
    cos = dot / (qn * kn)

    temp = jnp.clip(jnp.abs(scal_ref[0]), 0.1, 2.0)
    logits = (1.0 - cos + scal_ref[1]) / temp
    p = jax.nn.sigmoid(logits)

    shp = (1, BLK // 128, 128)
    pw = jnp.reshape(p, shp)
    gid = (b * BLK
           + lax.broadcasted_iota(jnp.int32, shp, 1) * 128
           + lax.broadcasted_iota(jnp.int32, shp, 2))
    force = jnp.logical_or(gid == 0, ov_ref[...] > 0.0)
    p_ref[...] = jnp.where(force, 1.0, pw)


@functools.partial(jax.jit, static_argnames=())
def kernel(hidden_states, cu_seqlens, Wq, Wk, temperature, boundary_bias):
    T, D = hidden_states.shape
    grid = (T // BLK,)
    scal = jnp.stack([temperature.astype(jnp.float32),
                      boundary_bias.astype(jnp.float32)])
    override = _build_override(cu_seqlens, T).reshape(T // BLK,
                                                      BLK // 128, 128)
    p_wide = pl.pallas_call(
        _routing_block,
        grid=grid,
        in_specs=[
            pl.BlockSpec(memory_space=pltpu.SMEM),
            pl.BlockSpec((BLK, D), lambda i: (i, 0)),
            pl.BlockSpec((D, D), lambda i: (0, 0)),
            pl.BlockSpec((D, D), lambda i: (0, 0)),
            pl.BlockSpec((1, BLK // 128, 128),
                         lambda i: (i, 0, 0)),
        ],
        out_specs=pl.BlockSpec((1, BLK // 128, 128), lambda i: (i, 0, 0)),
        out_shape=jax.ShapeDtypeStruct((T // BLK, BLK // 128, 128),
                                       jnp.float32),
        scratch_shapes=[pltpu.VMEM((8, D), jnp.float32)],
        compiler_params=pltpu.CompilerParams(
            dimension_semantics=("arbitrary",),
        ),
    )(scal, hidden_states, Wq, Wk, override)
    p = p_wide.reshape(T)
    one_m = 1.0 - p
    bp = jnp.stack([one_m, p], axis=-1)
    mask = p > 0.5
    sp = jnp.where(mask, p, one_m).reshape(T, 1)
    return bp, mask, sp


# speedup vs baseline: 1.0306x; 1.0306x over previous
import functools

import jax
import jax.numpy as jnp
from jax import lax
from jax.experimental import pallas as pl
from jax.experimental.pallas import tpu as pltpu
from jax.experimental.pallas import tpu_sc as plsc

BLK = 1024

_SC_WORKERS = 32


def _sc_override(cu_ref, out_ref, o_v, idx_v):
    slc = o_v.shape[0]
    wid = lax.axis_index("s") * 2 + lax.axis_index("c")
    base = wid * slc
    pltpu.sync_copy(cu_ref.at[pl.ds(0, 16)], idx_v)
    zeros16 = jnp.zeros((16,), jnp.float32)
    for i in range(slc // 16):
        o_v[pl.ds(i * 16, 16)] = zeros16
    idx = idx_v[...]
    in_range = jnp.logical_and(idx >= base, idx < base + slc)
    loc = jnp.where(in_range, idx - base, 0)
    plsc.store_scatter(o_v, [loc], jnp.full((16,), 1.0, jnp.float32),
                       mask=in_range)
    pltpu.sync_copy(o_v, out_ref.at[pl.ds(base, slc)])


def _build_override(cu_seqlens, T):
    mesh = plsc.VectorSubcoreMesh(core_axis_name="c", subcore_axis_name="s")
    fn = pl.kernel(
        _sc_override,
        out_type=jax.ShapeDtypeStruct((T,), jnp.float32),
        mesh=mesh,
        scratch_types=[
            pltpu.VMEM((T // _SC_WORKERS,), jnp.float32),
            pltpu.VMEM((16,), jnp.int32),
        ],
        compiler_params=pltpu.CompilerParams(needs_layout_passes=False),
    )
    return fn(cu_seqlens)


def _routing_block(scal_ref, hs_ref, wq_ref, wk_ref, ov_ref,
                   p_ref, carry_ref):
    b = pl.program_id(0)
    cur = hs_ref[...]

    dims = (((1,), (1,)), ((), ()))
    a = lax.dot_general(cur, wq_ref[...], dims,
                        preferred_element_type=jnp.float32)
    k = lax.dot_general(cur, wk_ref[...], dims,
                        preferred_element_type=jnp.float32)

    prev_last = carry_ref[7:8, :]
    carry_ref[...] = a[BLK - 8:, :]
    row_iota = lax.broadcasted_iota(jnp.int32, (BLK, 1), 0)
    q = jnp.where(row_iota == 0, prev_last, pltpu.roll(a, shift=1, axis=0))

    dot = jnp.sum(q * k, axis=1, keepdims=True)
    qn = jnp.maximum(jnp.sqrt(jnp.sum(q * q, axis=1, keepdims=True)), 1e-12)
    kn = jnp.maximum(jnp.sqrt(jnp.sum(k * k, axis=1, keepdims=True)), 1e-12)
    cos = dot / (qn * kn)

    temp = jnp.clip(jnp.abs(scal_ref[0]), 0.1, 2.0)
    logits = (1.0 - cos + scal_ref[1]) / temp
    p = jax.nn.sigmoid(logits)

    shp = (1, BLK // 128, 128)
    pw = jnp.reshape(p, shp)
    gid = (b * BLK
           + lax.broadcasted_iota(jnp.int32, shp, 1) * 128
           + lax.broadcasted_iota(jnp.int32, shp, 2))
    force = jnp.logical_or(gid == 0, ov_ref[...] > 0.0)
    p_ref[...] = jnp.where(force, 1.0, pw)


@functools.partial(jax.jit, static_argnames=())
def kernel(hidden_states, cu_seqlens, Wq, Wk, temperature, boundary_bias):
    T, D = hidden_states.shape
    grid = (T // BLK,)
    scal = jnp.stack([temperature.astype(jnp.float32),
                      boundary_bias.astype(jnp.float32)])
    override = _build_override(cu_seqlens, T).reshape(T // BLK,
                                                      BLK // 128, 128)
    p_wide = pl.pallas_call(
        _routing_block,
        grid=grid,
        in_specs=[
            pl.BlockSpec(memory_space=pltpu.SMEM),
            pl.BlockSpec((BLK, D), lambda i: (i, 0)),
            pl.BlockSpec((D, D), lambda i: (0, 0)),
            pl.BlockSpec((D, D), lambda i: (0, 0)),
            pl.BlockSpec((1, BLK // 128, 128),
                         lambda i: (i, 0, 0)),
        ],
        out_specs=pl.BlockSpec((1, BLK // 128, 128), lambda i: (i, 0, 0)),
        out_shape=jax.ShapeDtypeStruct((T // BLK, BLK // 128, 128),
                                       jnp.float32),
        scratch_shapes=[pltpu.VMEM((8, D), jnp.float32)],
        compiler_params=pltpu.CompilerParams(
            dimension_semantics=("arbitrary",),
        ),
    )(scal, hidden_states, Wq, Wk, override)
    p = p_wide.reshape(T)
    one_m = 1.0 - p
    bp = jnp.stack([one_m, p], axis=-1)
    mask = p > 0.5
    sp = jnp.where(mask, p, one_m).reshape(T, 1)
    return bp, mask, sp
